# Initial kernel scaffold; baseline (speedup 1.0000x reference)
import jax
import jax.numpy as jnp
from jax.experimental import pallas as pl


def kernel(x, edge_index, batch, W1, a1_src, a1_dst, b1, g1, beta1, W2, a2_src, a2_dst, b2, g2, beta2, Wc1, bc1, Wc2, bc2):
    raise NotImplementedError("write your pallas kernel here")


# speedup vs baseline: 53.4613x; 53.4613x over previous
import jax
import jax.numpy as jnp
from jax import lax
from jax.experimental import pallas as pl
from jax.experimental.pallas import tpu as pltpu
from jax.experimental.pallas import tpu_sc as plsc

N = 10000
E = 320000
G = 256
NC = 2
NS = 16
L = 16
CH = 128
NCH = E // CH
CPW = -(-NCH // (NC * NS))
NP = 10240
RPT = NP // NS
AW = 40
CPW2 = -(-NCH // NS)


def _leaky_exp(e):
    return jnp.exp(jnp.where(e > 0, e, 0.2 * e))


def _iota16():
    return lax.iota(jnp.int32, L)


def _full16(v):
    return jnp.full((L,), v, jnp.int32)


def _unpack_edge(w):
    return lax.shift_right_logical(w, 14), lax.bitwise_and(w, 16383)


def _k1_body(x_hbm, pck_hbm, sdb_hbm, out_hbm,
             xv, sdbv, pbuf, div, stg, acc):
    c = lax.axis_index("c")
    s = lax.axis_index("s")
    w = c * NS + s

    pltpu.sync_copy(x_hbm, xv)
    pltpu.sync_copy(sdb_hbm, sdbv)

    z16 = jnp.zeros((L,), jnp.float32)

    @pl.loop(0, CH // 2)
    def _zs(r):
        r2 = _full16(2 * r) + lax.shift_right_logical(_iota16(), 3)
        c2 = lax.bitwise_and(_iota16(), 7)
        plsc.store_scatter(stg, [r2, c2], z16)

    for j in range(RPT // CH):
        pltpu.sync_copy(stg, acc.at[pl.ds(s * RPT + j * CH, CH)])
    plsc.subcore_barrier()

    lo = jnp.minimum(w * CPW, NCH)
    hi = jnp.minimum(lo + CPW, NCH)

    @pl.loop(lo, hi)
    def _chunk(cid):
        base = cid * CH
        pltpu.sync_copy(pck_hbm.at[pl.ds(base, CH)], pbuf)
        for gl in range(CH // L):
            si, di = _unpack_edge(pbuf[pl.ds(gl * L, L)])
            div[pl.ds(gl * L, L)] = di
            xs = plsc.load_gather(xv, [si])
            xd = plsc.load_gather(xv, [di])
            rows = _iota16() + gl * L
            for h in range(4):
                ex = _leaky_exp(xs * sdbv[h] + xd * sdbv[h + 4])
                plsc.store_scatter(stg, [rows, _full16(h)], ex * xs)
                plsc.store_scatter(stg, [rows, _full16(h + 4)], ex)
        pltpu.sync_copy(stg, acc.at[div], add=True)

    plsc.subcore_barrier()
    pltpu.sync_copy(acc.at[pl.ds(s * RPT, RPT)],
                    out_hbm.at[c, pl.ds(s * RPT, RPT)])


def _k1(xf, pck, sdb):
    mesh = plsc.VectorSubcoreMesh(core_axis_name="c", subcore_axis_name="s",
                                  num_cores=NC, num_subcores=NS)
    return pl.kernel(
        _k1_body,
        out_type=jax.ShapeDtypeStruct((NC, NP, 8), jnp.float32),
        mesh=mesh,
        compiler_params=pltpu.CompilerParams(
            needs_layout_passes=False, use_tc_tiling_on_sc=False),
        scratch_types=[
            pltpu.VMEM((N,), jnp.float32),
            pltpu.VMEM((8, L), jnp.float32),
            pltpu.VMEM((CH,), jnp.int32),
            pltpu.VMEM((CH,), jnp.int32),
            pltpu.VMEM((CH, 8), jnp.float32),
            pltpu.VMEM_SHARED((NP, 8), jnp.float32),
        ],
    )(xf, pck, sdb)


def _k3_body(xp2b_hbm, sd_hbm, pck_hbm, outm_hbm,
             sdv, pbuf, siv, div, grow, mstg, accm, sem):
    c = lax.axis_index("c")
    s = lax.axis_index("s")

    pltpu.sync_copy(sd_hbm, sdv)

    z16 = jnp.zeros((L,), jnp.float32)

    @pl.loop(0, CH)
    def _zm(r):
        plsc.store_scatter(mstg, [_full16(r), _iota16()], z16)
        plsc.store_scatter(mstg, [_full16(r), _iota16() + L], z16)
        plsc.store_scatter(mstg, [_full16(r), _iota16() + (AW - L)], z16)

    for j in range(RPT // CH):
        pltpu.sync_copy(mstg, accm.at[pl.ds(s * RPT + j * CH, CH)])
    plsc.subcore_barrier()

    lo = jnp.minimum(s * CPW2, NCH)
    hi = jnp.minimum(lo + CPW2, NCH)
    himask = jnp.int32(0xFFFF0000 - 2**32)
    colc = _full16(0) + c
    shamt = _full16(16) - 16 * c

    @pl.loop(lo, hi)
    def _chunk(cid):
        base = cid * CH
        pltpu.sync_copy(pck_hbm.at[pl.ds(base, CH)], pbuf)
        for gl in range(CH // L):
            si, di = _unpack_edge(pbuf[pl.ds(gl * L, L)])
            siv[pl.ds(gl * L, L)] = si
            div[pl.ds(gl * L, L)] = di
            ex = _leaky_exp(plsc.load_gather(sdv, [si, colc])
                            + plsc.load_gather(sdv, [di, colc + 2]))
            rows = _iota16() + gl * L
            plsc.store_scatter(mstg, [rows, _full16(32)], ex)
        pltpu.async_copy(xp2b_hbm.at[siv], grow, sem).wait()
        for gl in range(CH // L):
            rows = _iota16() + gl * L
            ex = plsc.load_gather(mstg, [rows, _full16(32)])
            for cc in range(32):
                word = plsc.load_gather(grow, [rows, _full16(cc)])
                v = plsc.bitcast(lax.bitwise_and(
                    lax.shift_left(word, shamt), himask), jnp.float32)
                plsc.store_scatter(mstg, [rows, _full16(cc)], v * ex)
        pltpu.sync_copy(mstg, accm.at[div], add=True)

    plsc.subcore_barrier()
    pltpu.sync_copy(accm.at[pl.ds(s * RPT, RPT)],
                    outm_hbm.at[c, pl.ds(s * RPT, RPT)])


def _k3(xp2b, sd, pck):
    mesh = plsc.VectorSubcoreMesh(core_axis_name="c", subcore_axis_name="s",
                                  num_cores=NC, num_subcores=NS)
    return pl.kernel(
        _k3_body,
        out_type=jax.ShapeDtypeStruct((NC, NP, AW), jnp.float32),
        mesh=mesh,
        compiler_params=pltpu.CompilerParams(
            needs_layout_passes=False, use_tc_tiling_on_sc=False),
        scratch_types=[
            pltpu.VMEM((N, 4), jnp.float32),
            pltpu.VMEM((CH,), jnp.int32),
            pltpu.VMEM((CH,), jnp.int32),
            pltpu.VMEM((CH,), jnp.int32),
            pltpu.VMEM((CH, 32), jnp.int32),
            pltpu.VMEM((CH, AW), jnp.float32),
            pltpu.VMEM_SHARED((NP, AW), jnp.float32),
            pltpu.SemaphoreType.DMA,
        ],
    )(xp2b, sd, pck)


def _k2_body(p1_ref, x_ref, sdsum_ref, b1mat_ref, g1_ref, beta1_ref,
             w2_ref, a2s_ref, a2d_ref, xp2_ref, xp2b_ref, sd_ref):
    num = p1_ref[0, :N, 0:4] + p1_ref[1, :N, 0:4]
    den = p1_ref[0, :N, 4:8] + p1_ref[1, :N, 4:8]
    xcol = x_ref[...]
    es = _leaky_exp(xcol * sdsum_ref[...])
    num = num + es * xcol
    den = den + es
    r = num / (den + 1e-16)
    h1 = jnp.dot(r, b1mat_ref[...], preferred_element_type=jnp.float32)
    mu = jnp.mean(h1, axis=0, keepdims=True)
    var = jnp.mean((h1 - mu) ** 2, axis=0, keepdims=True)
    hb = (h1 - mu) / jnp.sqrt(var + 1e-5) * g1_ref[...] + beta1_ref[...]
    act = jnp.where(hb > 0, hb, jnp.exp(hb) - 1.0)
    xp2 = jnp.dot(act, w2_ref[...], preferred_element_type=jnp.float32)
    xp2_ref[...] = xp2
    ulo = lax.bitcast_convert_type(xp2[:, 0:32], jnp.uint32) + 0x8000
    uhi = lax.bitcast_convert_type(xp2[:, 32:64], jnp.uint32) + 0x8000
    packed = jnp.bitwise_or(jnp.bitwise_and(uhi, jnp.uint32(0xFFFF0000)),
                            jnp.right_shift(ulo, jnp.uint32(16)))
    xp2b_ref[...] = lax.bitcast_convert_type(packed, jnp.int32)
    s2 = jnp.dot(xp2, a2s_ref[...], preferred_element_type=jnp.float32)
    d2 = jnp.dot(xp2, a2d_ref[...], preferred_element_type=jnp.float32)
    sd_ref[...] = jnp.concatenate([s2, d2], axis=1)


def _k2(p1, xcol, sdsum, b1mat, g1, beta1, W2, a2s, a2d):
    return pl.pallas_call(
        _k2_body,
        out_shape=(jax.ShapeDtypeStruct((N, 64), jnp.float32),
                   jax.ShapeDtypeStruct((N, 32), jnp.int32),
                   jax.ShapeDtypeStruct((N, 4), jnp.float32)),
    )(p1, xcol, sdsum, b1mat, g1, beta1, W2, a2s, a2d)


def _k4_body(pm_ref, xp2_ref, sd_ref, b2_ref, g2_ref, beta2_ref,
             batch_ref, wc1_ref, bc1_ref, wc2_ref, bc2_ref,
             y_ref, pooled_ref):
    msg = jnp.concatenate([pm_ref[0, :N, 0:32], pm_ref[1, :N, 0:32]],
                          axis=1)
    den = jnp.concatenate([pm_ref[0, :N, 32:33], pm_ref[1, :N, 32:33]],
                          axis=1)
    ex = _leaky_exp(sd_ref[:, 0:2] + sd_ref[:, 2:4])
    xp2 = xp2_ref[...]
    exb = jnp.concatenate([jnp.broadcast_to(ex[:, 0:1], (N, 32)),
                           jnp.broadcast_to(ex[:, 1:2], (N, 32))], axis=1)
    msg = msg + xp2 * exb
    den = den + ex
    denb = jnp.concatenate([jnp.broadcast_to(den[:, 0:1], (N, 32)),
                            jnp.broadcast_to(den[:, 1:2], (N, 32))], axis=1)
    h2 = msg / (denb + 1e-16) + b2_ref[...]
    mu = jnp.mean(h2, axis=0, keepdims=True)
    var = jnp.mean((h2 - mu) ** 2, axis=0, keepdims=True)
    hb = (h2 - mu) / jnp.sqrt(var + 1e-5) * g2_ref[...] + beta2_ref[...]
    act = jnp.where(hb > 0, hb, jnp.exp(hb) - 1.0)
    gid = lax.broadcasted_iota(jnp.int32, (G, N), 0)
    oht = jnp.where(gid == batch_ref[...], 1.0, 0.0)
    sums = jnp.dot(oht, act, preferred_element_type=jnp.float32)
    cnt = jnp.sum(oht, axis=1, keepdims=True)
    pooled = sums / jnp.maximum(cnt, 1.0)
    hid = jnp.maximum(
        jnp.dot(pooled, wc1_ref[...], preferred_element_type=jnp.float32)
        + bc1_ref[...], 0.0)
    y = jnp.dot(hid, wc2_ref[...], preferred_element_type=jnp.float32) \
        + bc2_ref[...]
    y_ref[...] = y[:, 0]
    pooled_ref[...] = pooled


def _k4(pm, xp2, sd, b2, g2, beta2, batchrow, Wc1, bc1, Wc2, bc2):
    return pl.pallas_call(
        _k4_body,
        out_shape=(jax.ShapeDtypeStruct((G,), jnp.float32),
                   jax.ShapeDtypeStruct((G, 64), jnp.float32)),
    )(pm, xp2, sd, b2, g2, beta2, batchrow, Wc1, bc1, Wc2, bc2)


def kernel(x, edge_index, batch, W1, a1_src, a1_dst, b1, g1, beta1,
           W2, a2_src, a2_dst, b2, g2, beta2, Wc1, bc1, Wc2, bc2):
    xf = x[:, 0]
    pck = edge_index[0] * jnp.int32(16384) + edge_index[1]

    W1r = W1.reshape(4, 64)
    S1 = jnp.sum(W1r * a1_src, axis=1)
    D1 = jnp.sum(W1r * a1_dst, axis=1)
    sdb = jnp.concatenate([jnp.broadcast_to(S1[:, None], (4, L)),
                           jnp.broadcast_to(D1[:, None], (4, L))], axis=0)
    b1mat = (jnp.eye(4, dtype=jnp.float32)[:, :, None]
             * W1r[None, :, :]).reshape(4, 256)
    a2s = jnp.zeros((64, 2), jnp.float32)
    a2s = a2s.at[0:32, 0].set(a2_src[0]).at[32:64, 1].set(a2_src[1])
    a2d = jnp.zeros((64, 2), jnp.float32)
    a2d = a2d.at[0:32, 0].set(a2_dst[0]).at[32:64, 1].set(a2_dst[1])

    p1 = _k1(xf, pck, sdb)

    xp2, xp2b, sd = _k2(p1, x, (S1 + D1).reshape(1, 4), b1mat,
                        g1.reshape(1, 256), beta1.reshape(1, 256),
                        W2, a2s, a2d)

    pm = _k3(xp2b, sd, pck)

    y, pooled = _k4(pm, xp2, sd, b2.reshape(1, 64), g2.reshape(1, 64),
                    beta2.reshape(1, 64), batch.reshape(1, N),
                    Wc1, bc1.reshape(1, 32), Wc2, bc2.reshape(1, 1))
    return (y, pooled)
